# baseline (speedup 1.0000x reference)
import functools

import jax
import jax.numpy as jnp
from jax import lax
from jax.experimental import pallas as pl
from jax.experimental.pallas import tpu as pltpu
from jax.experimental.pallas import tpu_sc as plsc

_VOCAB = 100000
_EMBED_DIM = 64
_BATCH = 4096
_HIST = 50

_NC = 2
_NS = 16
_NW = _NC * _NS

_TOTAL = _BATCH * _HIST
_PER_W = _TOTAL // _NW
_IW = 128
_ROWS_PER_W = _PER_W // _IW
_GROUP = 5
_NSTEP = _ROWS_PER_W // _GROUP
_CHUNK = _GROUP * _IW


def _gather_body(table_hbm, idx_hbm, out_hbm, idx_v, buf0, buf1,
                 sem_g0, sem_g1, sem_w0, sem_w1):
  wid = lax.axis_index("s") * _NC + lax.axis_index("c")
  base = wid * _PER_W
  pltpu.sync_copy(idx_hbm.at[wid], idx_v)

  def fire_gathers(step, buf, sem):
    return [
        pltpu.async_copy(
            table_hbm.at[idx_v.at[step * _GROUP + b]],
            buf.at[pl.ds(b * _IW, _IW)],
            sem,
        ) for b in range(_GROUP)
    ]

  def superstep(i, carry):
    s0 = 2 * i
    s1 = 2 * i + 1

    @pl.when(i > 0)
    def _wait_prev_writes():
      pltpu.make_async_copy(
          buf0, out_hbm.at[pl.ds(base + (s0 - 2) * _CHUNK, _CHUNK)],
          sem_w0).wait()
      pltpu.make_async_copy(
          buf1, out_hbm.at[pl.ds(base + (s1 - 2) * _CHUNK, _CHUNK)],
          sem_w1).wait()

    g0 = fire_gathers(s0, buf0, sem_g0)
    g1 = fire_gathers(s1, buf1, sem_g1)
    for c in g0:
      c.wait()
    pltpu.async_copy(buf0, out_hbm.at[pl.ds(base + s0 * _CHUNK, _CHUNK)],
                     sem_w0)
    for c in g1:
      c.wait()
    pltpu.async_copy(buf1, out_hbm.at[pl.ds(base + s1 * _CHUNK, _CHUNK)],
                     sem_w1)
    return carry

  lax.fori_loop(0, _NSTEP // 2, superstep, 0)
  pltpu.make_async_copy(
      buf0, out_hbm.at[pl.ds(base + (_NSTEP - 2) * _CHUNK, _CHUNK)],
      sem_w0).wait()
  pltpu.make_async_copy(
      buf1, out_hbm.at[pl.ds(base + (_NSTEP - 1) * _CHUNK, _CHUNK)],
      sem_w1).wait()


@functools.partial(jax.jit, static_argnames=())
def kernel(indices, table):
  idx = indices.reshape(-1).astype(jnp.int32).reshape(_NW, _ROWS_PER_W, _IW)
  mesh = plsc.VectorSubcoreMesh(core_axis_name="c", subcore_axis_name="s")
  out = pl.kernel(
      _gather_body,
      out_type=jax.ShapeDtypeStruct((_TOTAL, _EMBED_DIM), jnp.float32),
      mesh=mesh,
      scratch_types=[
          pltpu.VMEM((_ROWS_PER_W, _IW), jnp.int32),
          pltpu.VMEM((_CHUNK, _EMBED_DIM), jnp.float32),
          pltpu.VMEM((_CHUNK, _EMBED_DIM), jnp.float32),
          pltpu.SemaphoreType.DMA,
          pltpu.SemaphoreType.DMA,
          pltpu.SemaphoreType.DMA,
          pltpu.SemaphoreType.DMA,
      ],
      compiler_params=pltpu.CompilerParams(use_tc_tiling_on_sc=False),
  )(table, idx)
  return out.reshape(_BATCH, _HIST, _EMBED_DIM)


# speedup vs baseline: 1.0034x; 1.0034x over previous
import jax
import jax.numpy as jnp
from jax import lax
from jax.experimental import pallas as pl
from jax.experimental.pallas import tpu as pltpu
from jax.experimental.pallas import tpu_sc as plsc

_VOCAB = 100000
_EMBED_DIM = 64
_BATCH = 4096
_HIST = 50

_NC = 2
_NS = 16
_NW = _NC * _NS

_B_PER_W = _BATCH // _NW
_NB = 16
_NSTEP = _B_PER_W // _NB


def _gather_body(table_hbm, idx_hbm, out_hbm, idx_v, buf0, buf1,
                 sem_g0, sem_g1, sem_w0, sem_w1):
  wid = lax.axis_index("s") * _NC + lax.axis_index("c")
  base = wid * _B_PER_W
  pltpu.sync_copy(idx_hbm.at[pl.ds(base, _B_PER_W)], idx_v)

  def fire_gathers(step, buf, sem):
    return [
        pltpu.async_copy(
            table_hbm.at[idx_v.at[step * _NB + b]],
            buf.at[b],
            sem,
        ) for b in range(_NB)
    ]

  def superstep(i, carry):
    s0 = 2 * i
    s1 = 2 * i + 1

    @pl.when(i > 0)
    def _wait_prev_writes():
      pltpu.make_async_copy(
          buf0, out_hbm.at[pl.ds(base + (s0 - 2) * _NB, _NB)], sem_w0).wait()
      pltpu.make_async_copy(
          buf1, out_hbm.at[pl.ds(base + (s1 - 2) * _NB, _NB)], sem_w1).wait()

    g0 = fire_gathers(s0, buf0, sem_g0)
    g1 = fire_gathers(s1, buf1, sem_g1)
    for c in g0:
      c.wait()
    pltpu.async_copy(buf0, out_hbm.at[pl.ds(base + s0 * _NB, _NB)], sem_w0)
    for c in g1:
      c.wait()
    pltpu.async_copy(buf1, out_hbm.at[pl.ds(base + s1 * _NB, _NB)], sem_w1)
    return carry

  lax.fori_loop(0, _NSTEP // 2, superstep, 0)
  pltpu.make_async_copy(
      buf0, out_hbm.at[pl.ds(base + (_NSTEP - 2) * _NB, _NB)], sem_w0).wait()
  pltpu.make_async_copy(
      buf1, out_hbm.at[pl.ds(base + (_NSTEP - 1) * _NB, _NB)], sem_w1).wait()


@jax.jit
def kernel(indices, table):
  idx = indices.astype(jnp.int32)
  mesh = plsc.VectorSubcoreMesh(core_axis_name="c", subcore_axis_name="s")
  out = pl.kernel(
      _gather_body,
      out_type=jax.ShapeDtypeStruct((_BATCH, _HIST, _EMBED_DIM), jnp.float32),
      mesh=mesh,
      scratch_types=[
          pltpu.VMEM((_B_PER_W, _HIST), jnp.int32),
          pltpu.VMEM((_NB, _HIST, _EMBED_DIM), jnp.float32),
          pltpu.VMEM((_NB, _HIST, _EMBED_DIM), jnp.float32),
          pltpu.SemaphoreType.DMA,
          pltpu.SemaphoreType.DMA,
          pltpu.SemaphoreType.DMA,
          pltpu.SemaphoreType.DMA,
      ],
      compiler_params=pltpu.CompilerParams(use_tc_tiling_on_sc=False),
  )(table, idx)
  return out
